# Initial kernel scaffold; baseline (speedup 1.0000x reference)
import functools

import jax
import jax.numpy as jnp
from jax import lax
from jax.experimental import pallas as pl
from jax.experimental.pallas import tpu as pltpu
from jax.experimental.pallas import tpu_sc as plsc

N = 10000
E = 320000
IN = 128
OUT = 64
HEADS = 2
EF = E + N
NSUB = 16
NCORE = 2
BR = 64
NB = 324
CHUNK = NB * BR
EPAD = NSUB * CHUNK
NPT = N // NSUB
DIVC = 125


def _sc_gat_layer(dh):
    row = dh + 16

    def body(xp_hbm, tab_hbm, src_hbm, dst_hbm, out_hbm,
             tabv, srcb, dstb, pb, g0, g1, s0, s1, dbuf, obuf,
             agg_sh, gsem, ssem):
        c = lax.axis_index("c")
        t = lax.axis_index("s")
        gbufs = (g0, g1)
        sbufs = (s0, s1)

        pltpu.sync_copy(tab_hbm.at[c], tabv)
        pltpu.sync_copy(src_hbm.at[t], srcb)
        pltpu.sync_copy(dst_hbm.at[t], dstb)

        iota16 = lax.iota(jnp.int32, 16)
        zv = jnp.zeros((16,), jnp.float32)

        def zero_row(i, carry):
            for q in range(row // 16):
                dbuf[i, pl.ds(q * 16, 16)] = zv
            return carry
        lax.fori_loop(0, DIVC, zero_row, 0)
        for r in range(NPT // DIVC):
            pltpu.sync_copy(dbuf, agg_sh.at[pl.ds(t * NPT + r * DIVC, DIVC)])

        def zero_pad(i, carry):
            s0[i, pl.ds(dh, 16)] = zv
            s1[i, pl.ds(dh, 16)] = zv
            return carry
        lax.fori_loop(0, BR, zero_pad, 0)

        bias = c * N
        chunk_base = t * CHUNK

        def scalar_blk(j, carry):
            for q in range(BR // 16):
                sv = srcb[j, pl.ds(q * 16, 16)]
                dv = dstb[j, pl.ds(q * 16, 16)]
                a_s = plsc.load_gather(tabv, [sv * 2])
                a_d = plsc.load_gather(tabv, [dv * 2 + 1])
                e = a_s + a_d
                e = jnp.maximum(e, 0.2 * e)
                p = jnp.exp(e)
                gid = chunk_base + j * BR + q * 16
                p = jnp.where(gid + iota16 < EF, p, 0.0)
                pb[j, pl.ds(q * 16, 16)] = p
                srcb[j, pl.ds(q * 16, 16)] = sv + bias
            return carry
        lax.fori_loop(0, NB, scalar_blk, 0)

        plsc.subcore_barrier()

        def g_desc(j, b):
            return pltpu.make_async_copy(xp_hbm.at[srcb.at[j]], gbufs[b], gsem)

        def s_desc(j, b):
            return pltpu.make_async_copy(sbufs[b], agg_sh.at[dstb.at[j]], ssem)

        g_desc(0, 0).start()

        colv = jnp.full((16,), dh, jnp.int32)

        def heavy(i, carry):
            for b in range(2):
                j = 2 * i + b
                g_desc(j, b).wait()

                @pl.when(j + 1 < NB)
                def _():
                    g_desc(j + 1, 1 - b).start()

                @pl.when(j >= 2)
                def _():
                    s_desc(j - 2, b).wait()

                gb = gbufs[b]
                sb = sbufs[b]
                for q in range(BR // 16):
                    pv = pb[j, pl.ds(q * 16, 16)]
                    plsc.store_scatter(sb, [q * 16 + iota16, colv], pv)
                for k in range(BR):
                    pk = pb[j, k]
                    for cc in range(dh // 16):
                        sl = pl.ds(cc * 16, 16)
                        sb[k, sl] = gb[k, sl] * pk
                s_desc(j, b).start(add=True)
            return carry
        lax.fori_loop(0, NB // 2, heavy, 0)
        s_desc(NB - 2, 0).wait()
        s_desc(NB - 1, 1).wait()

        plsc.subcore_barrier()

        def div_row(n, carry):
            d = dbuf[n, dh]
            inv = 1.0 / jnp.maximum(jnp.full((16,), d), 1e-16)
            for cc in range(dh // 16):
                sl = pl.ds(cc * 16, 16)
                obuf[n, sl] = dbuf[n, sl] * inv
            return carry

        for r in range(NPT // DIVC):
            base = t * NPT + r * DIVC
            pltpu.sync_copy(agg_sh.at[pl.ds(base, DIVC)], dbuf)
            lax.fori_loop(0, DIVC, div_row, 0)
            pltpu.sync_copy(obuf, out_hbm.at[c].at[pl.ds(base, DIVC)])

    mesh = plsc.VectorSubcoreMesh(core_axis_name="c", subcore_axis_name="s")
    return pl.kernel(
        body,
        out_type=jax.ShapeDtypeStruct((NCORE, N, dh), jnp.float32),
        mesh=mesh,
        scratch_types=[
            pltpu.VMEM((2 * N,), jnp.float32),
            pltpu.VMEM((NB, BR), jnp.int32),
            pltpu.VMEM((NB, BR), jnp.int32),
            pltpu.VMEM((NB, BR), jnp.float32),
            pltpu.VMEM((BR, dh), jnp.float32),
            pltpu.VMEM((BR, dh), jnp.float32),
            pltpu.VMEM((BR, row), jnp.float32),
            pltpu.VMEM((BR, row), jnp.float32),
            pltpu.VMEM((DIVC, row), jnp.float32),
            pltpu.VMEM((DIVC, dh), jnp.float32),
            pltpu.VMEM_SHARED((N, row), jnp.float32),
            pltpu.SemaphoreType.DMA,
            pltpu.SemaphoreType.DMA,
        ],
        name=f"sc_gat_d{dh}",
    )


_sc_gat64 = _sc_gat_layer(64)
_sc_gat32 = _sc_gat_layer(32)


_BN = 500
_GRID = N // _BN


def _k1_body(x_ref, w1_ref, rwt_ref, rb_ref, amat_ref,
             xp_ref, id_ref, al_ref):
    xb = x_ref[...]
    xp = jnp.dot(xb, w1_ref[...], preferred_element_type=jnp.float32)
    xp_ref[0] = xp[:, :OUT]
    xp_ref[1] = xp[:, OUT:]
    id_ref[...] = (
        jnp.dot(xb, rwt_ref[...], preferred_element_type=jnp.float32)
        + rb_ref[...]
    )
    al_ref[...] = jnp.dot(xp, amat_ref[...], preferred_element_type=jnp.float32)


def _k1(x, w1, rwt, rb, amat):
    return pl.pallas_call(
        _k1_body,
        grid=(_GRID,),
        in_specs=[
            pl.BlockSpec((_BN, IN), lambda i: (i, 0)),
            pl.BlockSpec((IN, 2 * OUT), lambda i: (0, 0)),
            pl.BlockSpec((IN, 2 * OUT), lambda i: (0, 0)),
            pl.BlockSpec((1, 2 * OUT), lambda i: (0, 0)),
            pl.BlockSpec((2 * OUT, 8), lambda i: (0, 0)),
        ],
        out_specs=[
            pl.BlockSpec((2, _BN, OUT), lambda i: (0, i, 0)),
            pl.BlockSpec((_BN, 2 * OUT), lambda i: (i, 0)),
            pl.BlockSpec((_BN, 8), lambda i: (i, 0)),
        ],
        out_shape=[
            jax.ShapeDtypeStruct((2, N, OUT), jnp.float32),
            jax.ShapeDtypeStruct((N, 2 * OUT), jnp.float32),
            jax.ShapeDtypeStruct((N, 8), jnp.float32),
        ],
    )(x, w1, rwt, rb, amat)


def _stats_body(p_ref, out_ref, acc):
    i = pl.program_id(0)

    @pl.when(i == 0)
    def _():
        acc[...] = jnp.zeros_like(acc)

    pre = jnp.concatenate([p_ref[0], p_ref[1]], axis=-1)
    acc[0:1, :] += jnp.sum(pre, axis=0, keepdims=True)
    acc[1:2, :] += jnp.sum(pre * pre, axis=0, keepdims=True)

    @pl.when(i == pl.num_programs(0) - 1)
    def _():
        out_ref[...] = acc[...]


def _stats(parts, dh):
    return pl.pallas_call(
        _stats_body,
        grid=(_GRID,),
        in_specs=[pl.BlockSpec((2, _BN, dh), lambda i: (0, i, 0))],
        out_specs=pl.BlockSpec((2, 2 * dh), lambda i: (0, 0)),
        out_shape=jax.ShapeDtypeStruct((2, 2 * dh), jnp.float32),
        scratch_shapes=[pltpu.VMEM((2, 2 * dh), jnp.float32)],
    )(parts)


def _k2b_body(p_ref, sums_ref, b1_ref, g_ref, bb_ref, id_ref, w2_ref,
              a2_ref, xp2_ref, al2_ref):
    pre = jnp.concatenate([p_ref[0], p_ref[1]], axis=-1) + b1_ref[...]
    s1 = sums_ref[0:1, :]
    s2 = sums_ref[1:2, :]
    mu = s1 / N + b1_ref[...]
    var = s2 / N - (s1 / N) ** 2
    scale = g_ref[...] / jnp.sqrt(var + 1e-5)
    h = jnp.maximum(scale * (pre - mu) + bb_ref[...], 0.0) + id_ref[...]
    xp2 = jnp.dot(h, w2_ref[...], preferred_element_type=jnp.float32)
    xp2_ref[0] = xp2[:, : OUT // 2]
    xp2_ref[1] = xp2[:, OUT // 2:]
    al2_ref[...] = jnp.dot(xp2, a2_ref[...], preferred_element_type=jnp.float32)


def _k2b(parts, sums, b1, g, bb, idt, w2, a2mat):
    return pl.pallas_call(
        _k2b_body,
        grid=(_GRID,),
        in_specs=[
            pl.BlockSpec((2, _BN, OUT), lambda i: (0, i, 0)),
            pl.BlockSpec((2, 2 * OUT), lambda i: (0, 0)),
            pl.BlockSpec((1, 2 * OUT), lambda i: (0, 0)),
            pl.BlockSpec((1, 2 * OUT), lambda i: (0, 0)),
            pl.BlockSpec((1, 2 * OUT), lambda i: (0, 0)),
            pl.BlockSpec((_BN, 2 * OUT), lambda i: (i, 0)),
            pl.BlockSpec((2 * OUT, OUT), lambda i: (0, 0)),
            pl.BlockSpec((OUT, 8), lambda i: (0, 0)),
        ],
        out_specs=[
            pl.BlockSpec((2, _BN, OUT // 2), lambda i: (0, i, 0)),
            pl.BlockSpec((_BN, 8), lambda i: (i, 0)),
        ],
        out_shape=[
            jax.ShapeDtypeStruct((2, N, OUT // 2), jnp.float32),
            jax.ShapeDtypeStruct((N, 8), jnp.float32),
        ],
    )(parts, sums, b1, g, bb, idt, w2, a2mat)


def _k4b_body(p_ref, sums_ref, b2_ref, g_ref, bb_ref, out_ref):
    pre = jnp.concatenate([p_ref[0], p_ref[1]], axis=-1) + b2_ref[...]
    s1 = sums_ref[0:1, :]
    s2 = sums_ref[1:2, :]
    mu = s1 / N + b2_ref[...]
    var = s2 / N - (s1 / N) ** 2
    scale = g_ref[...] / jnp.sqrt(var + 1e-5)
    out_ref[...] = jnp.maximum(scale * (pre - mu) + bb_ref[...], 0.0)


def _k4b(parts, sums, b2, g, bb):
    return pl.pallas_call(
        _k4b_body,
        grid=(_GRID,),
        in_specs=[
            pl.BlockSpec((2, _BN, OUT // 2), lambda i: (0, i, 0)),
            pl.BlockSpec((2, OUT), lambda i: (0, 0)),
            pl.BlockSpec((1, OUT), lambda i: (0, 0)),
            pl.BlockSpec((1, OUT), lambda i: (0, 0)),
            pl.BlockSpec((1, OUT), lambda i: (0, 0)),
        ],
        out_specs=pl.BlockSpec((_BN, OUT), lambda i: (i, 0)),
        out_shape=jax.ShapeDtypeStruct((N, OUT), jnp.float32),
    )(parts, sums, b2, g, bb)


def kernel(x, edge_index, W1, a_src1, a_dst1, b1, bn1_g, bn1_b, res_W, res_b,
           W2, a_src2, a_dst2, b2, bn2_g, bn2_b):
    f32 = jnp.float32

    loop = jnp.arange(N, dtype=edge_index.dtype)
    src_f = jnp.concatenate([edge_index[0], loop])
    dst_f = jnp.concatenate([edge_index[1], loop])
    pad = jnp.zeros((EPAD - EF,), dtype=edge_index.dtype)
    src_r = jnp.concatenate([src_f, pad]).astype(jnp.int32).reshape(NSUB, NB, BR)
    dst_r = jnp.concatenate([dst_f, pad]).astype(jnp.int32).reshape(NSUB, NB, BR)

    amat1 = jnp.zeros((2 * OUT, 8), f32)
    amat1 = amat1.at[:OUT, 0].set(a_src1[0])
    amat1 = amat1.at[OUT:, 1].set(a_src1[1])
    amat1 = amat1.at[:OUT, 2].set(a_dst1[0])
    amat1 = amat1.at[OUT:, 3].set(a_dst1[1])
    amat2 = jnp.zeros((OUT, 8), f32)
    amat2 = amat2.at[:, 0].set(a_src2[0])
    amat2 = amat2.at[:, 1].set(a_dst2[0])

    xp_parts, identity, al1 = _k1(
        x, W1, res_W.T, res_b.reshape(1, -1), amat1)

    tab1 = jnp.stack([
        jnp.stack([al1[:, 0], al1[:, 2]], axis=1).reshape(-1),
        jnp.stack([al1[:, 1], al1[:, 3]], axis=1).reshape(-1),
    ])

    agg1 = _sc_gat64(xp_parts.reshape(2 * N, OUT), tab1, src_r, dst_r)

    sums1 = _stats(agg1, OUT)
    xp2_parts, al2 = _k2b(
        agg1, sums1, b1.reshape(1, -1), bn1_g.reshape(1, -1),
        bn1_b.reshape(1, -1), identity, W2, amat2)

    tab2_row = jnp.stack([al2[:, 0], al2[:, 1]], axis=1).reshape(-1)
    tab2 = jnp.stack([tab2_row, tab2_row])

    agg2 = _sc_gat32(xp2_parts.reshape(2 * N, OUT // 2), tab2, src_r, dst_r)

    sums2 = _stats(agg2, OUT // 2)
    return _k4b(agg2, sums2, b2.reshape(1, -1), bn2_g.reshape(1, -1),
                bn2_b.reshape(1, -1))


# speedup vs baseline: 46.7435x; 46.7435x over previous
import functools

import jax
import jax.numpy as jnp
from jax import lax
from jax.experimental import pallas as pl
from jax.experimental.pallas import tpu as pltpu
from jax.experimental.pallas import tpu_sc as plsc

N = 10000
E = 320000
IN = 128
OUT = 64
HEADS = 2
EF = E + N
NSUB = 16
NCORE = 2
BR = 64
NB = 324
CHUNK = NB * BR
EPAD = NSUB * CHUNK
CH = 80
NCHUNK = N // CH
ROUNDS = (NCHUNK + NSUB - 1) // NSUB


DH = 32
ROW = DH + 16


def _sc_gat_layer(npass):
    def body(xp_hbm, tab_hbm, src_hbm, dst_hbm, out_hbm,
             tabv, srcb, dstb, pb, g0, g1, s0, s1, dbuf, obuf,
             agg_sh, gsem, ssem):
        c = lax.axis_index("c")
        t = lax.axis_index("s")
        gbufs = (g0, g1)
        sbufs = (s0, s1)

        pltpu.sync_copy(tab_hbm.at[c], tabv)
        pltpu.sync_copy(src_hbm.at[t], srcb)
        pltpu.sync_copy(dst_hbm.at[t], dstb)

        iota16 = lax.iota(jnp.int32, 16)
        zv = jnp.zeros((16,), jnp.float32)
        colv = jnp.full((16,), DH, jnp.int32)

        def zero_pad(i, carry):
            s0[i, pl.ds(DH, 16)] = zv
            s1[i, pl.ds(DH, 16)] = zv
            return carry
        lax.fori_loop(0, BR, zero_pad, 0)

        bias = c * (npass * N)
        chunk_base = t * CHUNK

        def scalar_blk(j, carry):
            for q in range(BR // 16):
                sv = srcb[j, pl.ds(q * 16, 16)]
                dv = dstb[j, pl.ds(q * 16, 16)]
                a_s = plsc.load_gather(tabv, [sv * 2])
                a_d = plsc.load_gather(tabv, [dv * 2 + 1])
                e = a_s + a_d
                e = jnp.maximum(e, 0.2 * e)
                p = jnp.exp(e)
                gid = chunk_base + j * BR + q * 16
                p = jnp.where(gid + iota16 < EF, p, 0.0)
                pb[j, pl.ds(q * 16, 16)] = p
                srcb[j, pl.ds(q * 16, 16)] = sv + bias
            return carry
        lax.fori_loop(0, NB, scalar_blk, 0)

        def g_desc(j, b):
            return pltpu.make_async_copy(xp_hbm.at[srcb.at[j]], gbufs[b], gsem)

        def s_desc(j, b):
            return pltpu.make_async_copy(sbufs[b], agg_sh.at[dstb.at[j]], ssem)

        def zero_row(i, carry):
            for q in range(ROW // 16):
                dbuf[i, pl.ds(q * 16, 16)] = zv
            return carry

        def heavy(i, carry):
            for b in range(2):
                j = 2 * i + b
                g_desc(j, b).wait()

                @pl.when(j + 1 < NB)
                def _():
                    g_desc(j + 1, 1 - b).start()

                @pl.when(j >= 2)
                def _():
                    s_desc(j - 2, b).wait()

                gb = gbufs[b]
                sb = sbufs[b]
                for q in range(BR // 16):
                    pv = pb[j, pl.ds(q * 16, 16)]
                    plsc.store_scatter(sb, [q * 16 + iota16, colv], pv)
                    for kk in range(16):
                        k = q * 16 + kk
                        pk = pv[kk]
                        for cc in range(DH // 16):
                            sl = pl.ds(cc * 16, 16)
                            sb[k, sl] = gb[k, sl] * pk
                s_desc(j, b).start(add=True)
            return carry

        def div_row(n, carry):
            d = dbuf[n, pl.ds(DH, 16)][0]
            inv = 1.0 / jnp.maximum(jnp.full((16,), d), 1e-16)
            for cc in range(DH // 16):
                sl = pl.ds(cc * 16, 16)
                obuf[n, sl] = dbuf[n, sl] * inv
            return carry

        def bump_src(j, carry):
            for q in range(BR // 16):
                sl = pl.ds(q * 16, 16)
                srcb[j, sl] = srcb[j, sl] + N
            return carry

        for ps in range(npass):
            if ps > 0:
                lax.fori_loop(0, NB, bump_src, 0)

            lax.fori_loop(0, CH, zero_row, 0)
            for r in range(ROUNDS):
                cid = t + NSUB * r

                @pl.when(cid < NCHUNK)
                def _():
                    pltpu.sync_copy(dbuf, agg_sh.at[pl.ds(cid * CH, CH)])

            plsc.subcore_barrier()

            g_desc(0, 0).start()
            lax.fori_loop(0, NB // 2, heavy, 0)
            s_desc(NB - 2, 0).wait()
            s_desc(NB - 1, 1).wait()

            plsc.subcore_barrier()

            for r in range(ROUNDS):
                cid = t + NSUB * r

                @pl.when(cid < NCHUNK)
                def _():
                    pltpu.sync_copy(agg_sh.at[pl.ds(cid * CH, CH)], dbuf)
                    lax.fori_loop(0, CH, div_row, 0)
                    pltpu.sync_copy(
                        obuf,
                        out_hbm.at[c * npass + ps].at[pl.ds(cid * CH, CH)])

            if ps + 1 < npass:
                plsc.subcore_barrier()

    mesh = plsc.VectorSubcoreMesh(core_axis_name="c", subcore_axis_name="s")
    return pl.kernel(
        body,
        out_type=jax.ShapeDtypeStruct((NCORE * npass, N, DH), jnp.float32),
        mesh=mesh,
        scratch_types=[
            pltpu.VMEM((2 * N,), jnp.float32),
            pltpu.VMEM((NB, BR), jnp.int32),
            pltpu.VMEM((NB, BR), jnp.int32),
            pltpu.VMEM((NB, BR), jnp.float32),
            pltpu.VMEM((BR, DH), jnp.float32),
            pltpu.VMEM((BR, DH), jnp.float32),
            pltpu.VMEM((BR, ROW), jnp.float32),
            pltpu.VMEM((BR, ROW), jnp.float32),
            pltpu.VMEM((CH, ROW), jnp.float32),
            pltpu.VMEM((CH, DH), jnp.float32),
            pltpu.VMEM_SHARED((N, ROW), jnp.float32),
            pltpu.SemaphoreType.DMA,
            pltpu.SemaphoreType.DMA,
        ],
        compiler_params=pltpu.CompilerParams(
            needs_layout_passes=False, use_tc_tiling_on_sc=False),
        name=f"sc_gat_p{npass}",
    )


_sc_gat_l1 = _sc_gat_layer(2)
_sc_gat_l2 = _sc_gat_layer(1)


_BN = 400
_GRID = N // _BN


def _k1_body(x_ref, w1_ref, rwt_ref, rb_ref, amat_ref,
             xp_ref, id_ref, al_ref):
    xb = x_ref[...]
    xp = jnp.dot(xb, w1_ref[...], preferred_element_type=jnp.float32)
    for b in range(4):
        xp_ref[b] = xp[:, b * DH:(b + 1) * DH]
    id_ref[...] = (
        jnp.dot(xb, rwt_ref[...], preferred_element_type=jnp.float32)
        + rb_ref[...]
    )
    al_ref[...] = jnp.dot(xp, amat_ref[...], preferred_element_type=jnp.float32)


def _k1(x, w1, rwt, rb, amat):
    return pl.pallas_call(
        _k1_body,
        grid=(_GRID,),
        in_specs=[
            pl.BlockSpec((_BN, IN), lambda i: (i, 0)),
            pl.BlockSpec((IN, 2 * OUT), lambda i: (0, 0)),
            pl.BlockSpec((IN, 2 * OUT), lambda i: (0, 0)),
            pl.BlockSpec((1, 2 * OUT), lambda i: (0, 0)),
            pl.BlockSpec((2 * OUT, 8), lambda i: (0, 0)),
        ],
        out_specs=[
            pl.BlockSpec((4, _BN, DH), lambda i: (0, i, 0)),
            pl.BlockSpec((_BN, 2 * OUT), lambda i: (i, 0)),
            pl.BlockSpec((_BN, 8), lambda i: (i, 0)),
        ],
        out_shape=[
            jax.ShapeDtypeStruct((4, N, DH), jnp.float32),
            jax.ShapeDtypeStruct((N, 2 * OUT), jnp.float32),
            jax.ShapeDtypeStruct((N, 8), jnp.float32),
        ],
    )(x, w1, rwt, rb, amat)


def _stats_body(nparts, p_ref, out_ref, acc):
    i = pl.program_id(0)

    @pl.when(i == 0)
    def _():
        acc[...] = jnp.zeros_like(acc)

    pre = jnp.concatenate([p_ref[b] for b in range(nparts)], axis=-1)
    acc[0:1, :] += jnp.sum(pre, axis=0, keepdims=True)
    acc[1:2, :] += jnp.sum(pre * pre, axis=0, keepdims=True)

    @pl.when(i == pl.num_programs(0) - 1)
    def _():
        out_ref[...] = acc[...]


def _stats(parts):
    nparts = parts.shape[0]
    d = nparts * DH
    return pl.pallas_call(
        functools.partial(_stats_body, nparts),
        grid=(_GRID,),
        in_specs=[pl.BlockSpec((nparts, _BN, DH), lambda i: (0, i, 0))],
        out_specs=pl.BlockSpec((2, d), lambda i: (0, 0)),
        out_shape=jax.ShapeDtypeStruct((2, d), jnp.float32),
        scratch_shapes=[pltpu.VMEM((2, d), jnp.float32)],
    )(parts)


def _k2b_body(p_ref, sums_ref, b1_ref, g_ref, bb_ref, id_ref, w2_ref,
              a2_ref, xp2_ref, al2_ref):
    pre = jnp.concatenate([p_ref[b] for b in range(4)], axis=-1) + b1_ref[...]
    s1 = sums_ref[0:1, :]
    s2 = sums_ref[1:2, :]
    mu = s1 / N + b1_ref[...]
    var = s2 / N - (s1 / N) ** 2
    scale = g_ref[...] / jnp.sqrt(var + 1e-5)
    h = jnp.maximum(scale * (pre - mu) + bb_ref[...], 0.0) + id_ref[...]
    xp2 = jnp.dot(h, w2_ref[...], preferred_element_type=jnp.float32)
    xp2_ref[0] = xp2[:, :DH]
    xp2_ref[1] = xp2[:, DH:]
    al2_ref[...] = jnp.dot(xp2, a2_ref[...], preferred_element_type=jnp.float32)


def _k2b(parts, sums, b1, g, bb, idt, w2, a2mat):
    return pl.pallas_call(
        _k2b_body,
        grid=(_GRID,),
        in_specs=[
            pl.BlockSpec((4, _BN, DH), lambda i: (0, i, 0)),
            pl.BlockSpec((2, 2 * OUT), lambda i: (0, 0)),
            pl.BlockSpec((1, 2 * OUT), lambda i: (0, 0)),
            pl.BlockSpec((1, 2 * OUT), lambda i: (0, 0)),
            pl.BlockSpec((1, 2 * OUT), lambda i: (0, 0)),
            pl.BlockSpec((_BN, 2 * OUT), lambda i: (i, 0)),
            pl.BlockSpec((2 * OUT, OUT), lambda i: (0, 0)),
            pl.BlockSpec((OUT, 8), lambda i: (0, 0)),
        ],
        out_specs=[
            pl.BlockSpec((2, _BN, DH), lambda i: (0, i, 0)),
            pl.BlockSpec((_BN, 8), lambda i: (i, 0)),
        ],
        out_shape=[
            jax.ShapeDtypeStruct((2, N, DH), jnp.float32),
            jax.ShapeDtypeStruct((N, 8), jnp.float32),
        ],
    )(parts, sums, b1, g, bb, idt, w2, a2mat)


def _k4b_body(p_ref, sums_ref, b2_ref, g_ref, bb_ref, out_ref):
    pre = jnp.concatenate([p_ref[0], p_ref[1]], axis=-1) + b2_ref[...]
    s1 = sums_ref[0:1, :]
    s2 = sums_ref[1:2, :]
    mu = s1 / N + b2_ref[...]
    var = s2 / N - (s1 / N) ** 2
    scale = g_ref[...] / jnp.sqrt(var + 1e-5)
    out_ref[...] = jnp.maximum(scale * (pre - mu) + bb_ref[...], 0.0)


def _k4b(parts, sums, b2, g, bb):
    return pl.pallas_call(
        _k4b_body,
        grid=(_GRID,),
        in_specs=[
            pl.BlockSpec((2, _BN, DH), lambda i: (0, i, 0)),
            pl.BlockSpec((2, OUT), lambda i: (0, 0)),
            pl.BlockSpec((1, OUT), lambda i: (0, 0)),
            pl.BlockSpec((1, OUT), lambda i: (0, 0)),
            pl.BlockSpec((1, OUT), lambda i: (0, 0)),
        ],
        out_specs=pl.BlockSpec((_BN, OUT), lambda i: (i, 0)),
        out_shape=jax.ShapeDtypeStruct((N, OUT), jnp.float32),
    )(parts, sums, b2, g, bb)


def kernel(x, edge_index, W1, a_src1, a_dst1, b1, bn1_g, bn1_b, res_W, res_b,
           W2, a_src2, a_dst2, b2, bn2_g, bn2_b):
    f32 = jnp.float32

    loop = jnp.arange(N, dtype=edge_index.dtype)
    src_f = jnp.concatenate([edge_index[0], loop])
    dst_f = jnp.concatenate([edge_index[1], loop])
    pad = jnp.zeros((EPAD - EF,), dtype=edge_index.dtype)
    src_r = jnp.concatenate([src_f, pad]).astype(jnp.int32).reshape(NSUB, NB, BR)
    dst_r = jnp.concatenate([dst_f, pad]).astype(jnp.int32).reshape(NSUB, NB, BR)

    amat1 = jnp.zeros((2 * OUT, 8), f32)
    amat1 = amat1.at[:OUT, 0].set(a_src1[0])
    amat1 = amat1.at[OUT:, 1].set(a_src1[1])
    amat1 = amat1.at[:OUT, 2].set(a_dst1[0])
    amat1 = amat1.at[OUT:, 3].set(a_dst1[1])
    amat2 = jnp.zeros((OUT, 8), f32)
    amat2 = amat2.at[:, 0].set(a_src2[0])
    amat2 = amat2.at[:, 1].set(a_dst2[0])

    xp_parts, identity, al1 = _k1(
        x, W1, res_W.T, res_b.reshape(1, -1), amat1)

    tab1 = jnp.stack([
        jnp.stack([al1[:, 0], al1[:, 2]], axis=1).reshape(-1),
        jnp.stack([al1[:, 1], al1[:, 3]], axis=1).reshape(-1),
    ])

    agg1 = _sc_gat_l1(xp_parts.reshape(4 * N, DH), tab1, src_r, dst_r)

    sums1 = _stats(agg1)
    xp2_parts, al2 = _k2b(
        agg1, sums1, b1.reshape(1, -1), bn1_g.reshape(1, -1),
        bn1_b.reshape(1, -1), identity, W2, amat2)

    tab2_row = jnp.stack([al2[:, 0], al2[:, 1]], axis=1).reshape(-1)
    tab2 = jnp.stack([tab2_row, tab2_row])

    agg2 = _sc_gat_l2(xp2_parts.reshape(2 * N, DH), tab2, src_r, dst_r)

    sums2 = _stats(agg2)
    return _k4b(agg2, sums2, b2.reshape(1, -1), bn2_g.reshape(1, -1),
                bn2_b.reshape(1, -1))


# speedup vs baseline: 65.6256x; 1.4040x over previous
import functools

import jax
import jax.numpy as jnp
from jax import lax
from jax.experimental import pallas as pl
from jax.experimental.pallas import tpu as pltpu
from jax.experimental.pallas import tpu_sc as plsc

N = 10000
E = 320000
IN = 128
OUT = 64
HEADS = 2
EF = E + N
NSUB = 16
NCORE = 2
BR = 64
NB = 324
CHUNK = NB * BR
EPAD = NSUB * CHUNK
CH = 80
NCHUNK = N // CH
ROUNDS = (NCHUNK + NSUB - 1) // NSUB


DH = 32
ROW = DH + 16


def _sc_gat_layer(npass):
    def body(xp_hbm, tab_hbm, src_hbm, dst_hbm, out_hbm,
             tabv, srcb, dstb, pb, g0, g1, g2, s0, s1,
             dbuf, obuf, agg_sh, gsem, ssem):
        c = lax.axis_index("c")
        t = lax.axis_index("s")
        gbufs = (g0, g1, g2)
        sbufs = (s0, s1)

        pltpu.sync_copy(tab_hbm.at[c], tabv)
        pltpu.sync_copy(src_hbm.at[t], srcb)
        pltpu.sync_copy(dst_hbm.at[t], dstb)

        iota16 = lax.iota(jnp.int32, 16)
        zv = jnp.zeros((16,), jnp.float32)
        colv = jnp.full((16,), DH, jnp.int32)

        def zero_pad(i, carry):
            for sb in sbufs:
                if ROW > DH:
                    sb[i, pl.ds(DH, 16)] = zv
            return carry
        lax.fori_loop(0, BR, zero_pad, 0)

        bias = c * (npass * N)
        chunk_base = t * CHUNK

        def scalar_blk(j, carry):
            for q in range(BR // 16):
                sv = srcb[j, pl.ds(q * 16, 16)]
                dv = dstb[j, pl.ds(q * 16, 16)]
                a_s = plsc.load_gather(tabv, [sv * 2])
                a_d = plsc.load_gather(tabv, [dv * 2 + 1])
                e = a_s + a_d
                e = jnp.maximum(e, 0.2 * e)
                p = jnp.exp(e)
                gid = chunk_base + j * BR + q * 16
                p = jnp.where(gid + iota16 < EF, p, 0.0)
                pb[j, pl.ds(q * 16, 16)] = p
                srcb[j, pl.ds(q * 16, 16)] = sv + bias
            return carry
        lax.fori_loop(0, NB, scalar_blk, 0)

        def g_desc(j, b):
            return pltpu.make_async_copy(xp_hbm.at[srcb.at[j]], gbufs[b], gsem)

        def s_desc(j, b):
            return pltpu.make_async_copy(sbufs[b], agg_sh.at[dstb.at[j]], ssem)

        def zero_row(i, carry):
            for q in range(ROW // 16):
                dbuf[i, pl.ds(q * 16, 16)] = zv
            return carry

        def heavy(i, carry):
            for b in range(6):
                j = 6 * i + b
                g_desc(j, b % 3).wait()

                @pl.when(j + 2 < NB)
                def _():
                    g_desc(j + 2, (b + 2) % 3).start()

                @pl.when(j >= 2)
                def _():
                    s_desc(j - 2, b % 2).wait()

                gb = gbufs[b % 3]
                sb = sbufs[b % 2]
                for q in range(BR // 16):
                    pv = pb[j, pl.ds(q * 16, 16)]
                    plsc.store_scatter(sb, [q * 16 + iota16, colv], pv)
                    for kk in range(16):
                        k = q * 16 + kk
                        pk = pv[kk]
                        for cc in range(DH // 16):
                            sl = pl.ds(cc * 16, 16)
                            sb[k, sl] = gb[k, sl] * pk
                s_desc(j, b % 2).start(add=True)
            return carry

        def div_row(n, carry):
            d = dbuf[n, pl.ds(ROW - 16, 16)][0]
            inv = 1.0 / jnp.maximum(jnp.full((16,), d), 1e-16)
            for cc in range(DH // 16):
                sl = pl.ds(cc * 16, 16)
                obuf[n, sl] = dbuf[n, sl] * inv
            return carry

        def bump_src(j, carry):
            for q in range(BR // 16):
                sl = pl.ds(q * 16, 16)
                srcb[j, sl] = srcb[j, sl] + N
            return carry

        for ps in range(npass):
            if ps > 0:
                lax.fori_loop(0, NB, bump_src, 0)

            lax.fori_loop(0, CH, zero_row, 0)
            for r in range(ROUNDS):
                cid = t + NSUB * r

                @pl.when(cid < NCHUNK)
                def _():
                    pltpu.sync_copy(dbuf, agg_sh.at[pl.ds(cid * CH, CH)])

            plsc.subcore_barrier()

            g_desc(0, 0).start()
            g_desc(1, 1).start()
            lax.fori_loop(0, NB // 6, heavy, 0)
            s_desc(NB - 2, 0).wait()
            s_desc(NB - 1, 1).wait()

            plsc.subcore_barrier()

            for r in range(ROUNDS):
                cid = t + NSUB * r

                @pl.when(cid < NCHUNK)
                def _():
                    pltpu.sync_copy(agg_sh.at[pl.ds(cid * CH, CH)], dbuf)
                    lax.fori_loop(0, CH, div_row, 0)
                    pltpu.sync_copy(
                        obuf,
                        out_hbm.at[c * npass + ps].at[pl.ds(cid * CH, CH)])

            if ps + 1 < npass:
                plsc.subcore_barrier()

    mesh = plsc.VectorSubcoreMesh(core_axis_name="c", subcore_axis_name="s")
    return pl.kernel(
        body,
        out_type=jax.ShapeDtypeStruct((NCORE * npass, N, DH), jnp.float32),
        mesh=mesh,
        scratch_types=[
            pltpu.VMEM((2 * N,), jnp.float32),
            pltpu.VMEM((NB, BR), jnp.int32),
            pltpu.VMEM((NB, BR), jnp.int32),
            pltpu.VMEM((NB, BR), jnp.float32),
            pltpu.VMEM((BR, DH), jnp.float32),
            pltpu.VMEM((BR, DH), jnp.float32),
            pltpu.VMEM((BR, DH), jnp.float32),
            pltpu.VMEM((BR, ROW), jnp.float32),
            pltpu.VMEM((BR, ROW), jnp.float32),
            pltpu.VMEM((CH, ROW), jnp.float32),
            pltpu.VMEM((CH, DH), jnp.float32),
            pltpu.VMEM_SHARED((N, ROW), jnp.float32),
            pltpu.SemaphoreType.DMA,
            pltpu.SemaphoreType.DMA,
        ],
        compiler_params=pltpu.CompilerParams(
            needs_layout_passes=False, use_tc_tiling_on_sc=False),
        name=f"sc_gat_p{npass}",
    )


_sc_gat_l1 = _sc_gat_layer(2)
_sc_gat_l2 = _sc_gat_layer(1)


_BN = 400
_GRID = N // _BN


def _k1_body(x_ref, w1_ref, rwt_ref, rb_ref, amat_ref,
             xp_ref, id_ref, al_ref):
    xb = x_ref[...]
    xp = jnp.dot(xb, w1_ref[...], preferred_element_type=jnp.float32)
    for b in range(4):
        xp_ref[b] = xp[:, b * DH:(b + 1) * DH]
    id_ref[...] = (
        jnp.dot(xb, rwt_ref[...], preferred_element_type=jnp.float32)
        + rb_ref[...]
    )
    al_ref[...] = jnp.dot(xp, amat_ref[...], preferred_element_type=jnp.float32)


def _k1(x, w1, rwt, rb, amat):
    return pl.pallas_call(
        _k1_body,
        grid=(_GRID,),
        in_specs=[
            pl.BlockSpec((_BN, IN), lambda i: (i, 0)),
            pl.BlockSpec((IN, 2 * OUT), lambda i: (0, 0)),
            pl.BlockSpec((IN, 2 * OUT), lambda i: (0, 0)),
            pl.BlockSpec((1, 2 * OUT), lambda i: (0, 0)),
            pl.BlockSpec((2 * OUT, 8), lambda i: (0, 0)),
        ],
        out_specs=[
            pl.BlockSpec((4, _BN, DH), lambda i: (0, i, 0)),
            pl.BlockSpec((_BN, 2 * OUT), lambda i: (i, 0)),
            pl.BlockSpec((_BN, 8), lambda i: (i, 0)),
        ],
        out_shape=[
            jax.ShapeDtypeStruct((4, N, DH), jnp.float32),
            jax.ShapeDtypeStruct((N, 2 * OUT), jnp.float32),
            jax.ShapeDtypeStruct((N, 8), jnp.float32),
        ],
    )(x, w1, rwt, rb, amat)


def _stats_body(nparts, p_ref, out_ref, acc):
    i = pl.program_id(0)

    @pl.when(i == 0)
    def _():
        acc[...] = jnp.zeros_like(acc)

    pre = jnp.concatenate([p_ref[b] for b in range(nparts)], axis=-1)
    acc[0:1, :] += jnp.sum(pre, axis=0, keepdims=True)
    acc[1:2, :] += jnp.sum(pre * pre, axis=0, keepdims=True)

    @pl.when(i == pl.num_programs(0) - 1)
    def _():
        out_ref[...] = acc[...]


def _stats(parts):
    nparts = parts.shape[0]
    d = nparts * DH
    return pl.pallas_call(
        functools.partial(_stats_body, nparts),
        grid=(_GRID,),
        in_specs=[pl.BlockSpec((nparts, _BN, DH), lambda i: (0, i, 0))],
        out_specs=pl.BlockSpec((2, d), lambda i: (0, 0)),
        out_shape=jax.ShapeDtypeStruct((2, d), jnp.float32),
        scratch_shapes=[pltpu.VMEM((2, d), jnp.float32)],
    )(parts)


def _k2b_body(p_ref, sums_ref, b1_ref, g_ref, bb_ref, id_ref, w2_ref,
              a2_ref, xp2_ref, al2_ref):
    pre = jnp.concatenate([p_ref[b] for b in range(4)], axis=-1) + b1_ref[...]
    s1 = sums_ref[0:1, :]
    s2 = sums_ref[1:2, :]
    mu = s1 / N + b1_ref[...]
    var = s2 / N - (s1 / N) ** 2
    scale = g_ref[...] / jnp.sqrt(var + 1e-5)
    h = jnp.maximum(scale * (pre - mu) + bb_ref[...], 0.0) + id_ref[...]
    xp2 = jnp.dot(h, w2_ref[...], preferred_element_type=jnp.float32)
    xp2_ref[0] = xp2[:, :DH]
    xp2_ref[1] = xp2[:, DH:]
    al2_ref[...] = jnp.dot(xp2, a2_ref[...], preferred_element_type=jnp.float32)


def _k2b(parts, sums, b1, g, bb, idt, w2, a2mat):
    return pl.pallas_call(
        _k2b_body,
        grid=(_GRID,),
        in_specs=[
            pl.BlockSpec((4, _BN, DH), lambda i: (0, i, 0)),
            pl.BlockSpec((2, 2 * OUT), lambda i: (0, 0)),
            pl.BlockSpec((1, 2 * OUT), lambda i: (0, 0)),
            pl.BlockSpec((1, 2 * OUT), lambda i: (0, 0)),
            pl.BlockSpec((1, 2 * OUT), lambda i: (0, 0)),
            pl.BlockSpec((_BN, 2 * OUT), lambda i: (i, 0)),
            pl.BlockSpec((2 * OUT, OUT), lambda i: (0, 0)),
            pl.BlockSpec((OUT, 8), lambda i: (0, 0)),
        ],
        out_specs=[
            pl.BlockSpec((2, _BN, DH), lambda i: (0, i, 0)),
            pl.BlockSpec((_BN, 8), lambda i: (i, 0)),
        ],
        out_shape=[
            jax.ShapeDtypeStruct((2, N, DH), jnp.float32),
            jax.ShapeDtypeStruct((N, 8), jnp.float32),
        ],
    )(parts, sums, b1, g, bb, idt, w2, a2mat)


def _k4b_body(p_ref, sums_ref, b2_ref, g_ref, bb_ref, out_ref):
    pre = jnp.concatenate([p_ref[0], p_ref[1]], axis=-1) + b2_ref[...]
    s1 = sums_ref[0:1, :]
    s2 = sums_ref[1:2, :]
    mu = s1 / N + b2_ref[...]
    var = s2 / N - (s1 / N) ** 2
    scale = g_ref[...] / jnp.sqrt(var + 1e-5)
    out_ref[...] = jnp.maximum(scale * (pre - mu) + bb_ref[...], 0.0)


def _k4b(parts, sums, b2, g, bb):
    return pl.pallas_call(
        _k4b_body,
        grid=(_GRID,),
        in_specs=[
            pl.BlockSpec((2, _BN, DH), lambda i: (0, i, 0)),
            pl.BlockSpec((2, OUT), lambda i: (0, 0)),
            pl.BlockSpec((1, OUT), lambda i: (0, 0)),
            pl.BlockSpec((1, OUT), lambda i: (0, 0)),
            pl.BlockSpec((1, OUT), lambda i: (0, 0)),
        ],
        out_specs=pl.BlockSpec((_BN, OUT), lambda i: (i, 0)),
        out_shape=jax.ShapeDtypeStruct((N, OUT), jnp.float32),
    )(parts, sums, b2, g, bb)


def kernel(x, edge_index, W1, a_src1, a_dst1, b1, bn1_g, bn1_b, res_W, res_b,
           W2, a_src2, a_dst2, b2, bn2_g, bn2_b):
    f32 = jnp.float32

    loop = jnp.arange(N, dtype=edge_index.dtype)
    src_f = jnp.concatenate([edge_index[0], loop])
    dst_f = jnp.concatenate([edge_index[1], loop])
    pad = jnp.zeros((EPAD - EF,), dtype=edge_index.dtype)
    src_r = jnp.concatenate([src_f, pad]).astype(jnp.int32).reshape(NSUB, NB, BR)
    dst_r = jnp.concatenate([dst_f, pad]).astype(jnp.int32).reshape(NSUB, NB, BR)

    amat1 = jnp.zeros((2 * OUT, 8), f32)
    amat1 = amat1.at[:OUT, 0].set(a_src1[0])
    amat1 = amat1.at[OUT:, 1].set(a_src1[1])
    amat1 = amat1.at[:OUT, 2].set(a_dst1[0])
    amat1 = amat1.at[OUT:, 3].set(a_dst1[1])
    amat2 = jnp.zeros((OUT, 8), f32)
    amat2 = amat2.at[:, 0].set(a_src2[0])
    amat2 = amat2.at[:, 1].set(a_dst2[0])

    xp_parts, identity, al1 = _k1(
        x, W1, res_W.T, res_b.reshape(1, -1), amat1)

    tab1 = jnp.stack([
        jnp.stack([al1[:, 0], al1[:, 2]], axis=1).reshape(-1),
        jnp.stack([al1[:, 1], al1[:, 3]], axis=1).reshape(-1),
    ])

    agg1 = _sc_gat_l1(xp_parts.reshape(4 * N, DH), tab1, src_r, dst_r)

    sums1 = _stats(agg1)
    xp2_parts, al2 = _k2b(
        agg1, sums1, b1.reshape(1, -1), bn1_g.reshape(1, -1),
        bn1_b.reshape(1, -1), identity, W2, amat2)

    tab2_row = jnp.stack([al2[:, 0], al2[:, 1]], axis=1).reshape(-1)
    tab2 = jnp.stack([tab2_row, tab2_row])

    agg2 = _sc_gat_l2(xp2_parts.reshape(2 * N, DH), tab2, src_r, dst_r)

    sums2 = _stats(agg2)
    return _k4b(agg2, sums2, b2.reshape(1, -1), bn2_g.reshape(1, -1),
                bn2_b.reshape(1, -1))


# speedup vs baseline: 73.4187x; 1.1188x over previous
import functools

import jax
import jax.numpy as jnp
from jax import lax
from jax.experimental import pallas as pl
from jax.experimental.pallas import tpu as pltpu
from jax.experimental.pallas import tpu_sc as plsc

N = 10000
E = 320000
IN = 128
OUT = 64
HEADS = 2
EF = E + N
NSUB = 16
NCORE = 2
BR = 64
NB = 324
CHUNK = NB * BR
EPAD = NSUB * CHUNK
CH = 40
NCHUNK = N // CH
ROUNDS = (NCHUNK + NSUB - 1) // NSUB


DH = 32
ROW = DH + 16


def _sc_gat_layer(npass):
    def body(xp_hbm, tab_hbm, src_hbm, dst_hbm, out_hbm,
             tabv, srcb, dstb, pb, g0, g1, g2, g3, s0, s1,
             dbuf, obuf, agg_sh, gsem, ssem):
        c = lax.axis_index("c")
        t = lax.axis_index("s")
        gbufs = (g0, g1, g2, g3)
        sbufs = (s0, s1)

        pltpu.sync_copy(tab_hbm.at[c], tabv)
        pltpu.sync_copy(src_hbm.at[t], srcb)
        pltpu.sync_copy(dst_hbm.at[t], dstb)

        iota16 = lax.iota(jnp.int32, 16)
        zv = jnp.zeros((16,), jnp.float32)
        colv = jnp.full((16,), DH, jnp.int32)

        def zero_pad(i, carry):
            for sb in sbufs:
                if ROW > DH:
                    sb[i, pl.ds(DH, 16)] = zv
            return carry
        lax.fori_loop(0, BR, zero_pad, 0)

        bias = c * (npass * N)
        chunk_base = t * CHUNK

        def scalar_blk(j, carry):
            for q in range(BR // 16):
                sv = srcb[j, pl.ds(q * 16, 16)]
                dv = dstb[j, pl.ds(q * 16, 16)]
                a_s = plsc.load_gather(tabv, [sv * 2])
                a_d = plsc.load_gather(tabv, [dv * 2 + 1])
                e = a_s + a_d
                e = jnp.maximum(e, 0.2 * e)
                p = jnp.exp(e)
                gid = chunk_base + j * BR + q * 16
                p = jnp.where(gid + iota16 < EF, p, 0.0)
                pb[j, pl.ds(q * 16, 16)] = p
                srcb[j, pl.ds(q * 16, 16)] = sv + bias
            return carry
        lax.fori_loop(0, NB, scalar_blk, 0)

        def g_desc(j, b):
            return pltpu.make_async_copy(xp_hbm.at[srcb.at[j]], gbufs[b], gsem)

        def s_desc(j, b):
            return pltpu.make_async_copy(sbufs[b], agg_sh.at[dstb.at[j]], ssem)

        def zero_row(i, carry):
            for q in range(ROW // 16):
                dbuf[i, pl.ds(q * 16, 16)] = zv
            return carry

        def heavy(i, carry):
            for b in range(4):
                j = 4 * i + b
                g_desc(j, b).wait()

                @pl.when(j + 3 < NB)
                def _():
                    g_desc(j + 3, (b + 3) % 4).start()

                @pl.when(j >= 2)
                def _():
                    s_desc(j - 2, b % 2).wait()

                gb = gbufs[b]
                sb = sbufs[b % 2]
                for q in range(BR // 16):
                    pv = pb[j, pl.ds(q * 16, 16)]
                    plsc.store_scatter(sb, [q * 16 + iota16, colv], pv)
                    for kk in range(16):
                        k = q * 16 + kk
                        pk = pv[kk]
                        for cc in range(DH // 16):
                            sl = pl.ds(cc * 16, 16)
                            sb[k, sl] = gb[k, sl] * pk
                s_desc(j, b % 2).start(add=True)
            return carry

        def div_row(n, carry):
            d = dbuf[n, pl.ds(ROW - 16, 16)][0]
            inv = 1.0 / jnp.maximum(jnp.full((16,), d), 1e-16)
            for cc in range(DH // 16):
                sl = pl.ds(cc * 16, 16)
                obuf[n, sl] = dbuf[n, sl] * inv
            return carry

        def bump_src(j, carry):
            for q in range(BR // 16):
                sl = pl.ds(q * 16, 16)
                srcb[j, sl] = srcb[j, sl] + N
            return carry

        for ps in range(npass):
            if ps > 0:
                lax.fori_loop(0, NB, bump_src, 0)

            lax.fori_loop(0, CH, zero_row, 0)
            for r in range(ROUNDS):
                cid = t + NSUB * r

                @pl.when(cid < NCHUNK)
                def _():
                    pltpu.sync_copy(dbuf, agg_sh.at[pl.ds(cid * CH, CH)])

            plsc.subcore_barrier()

            for b in range(3):
                g_desc(b, b).start()
            lax.fori_loop(0, NB // 4, heavy, 0)
            s_desc(NB - 2, 0).wait()
            s_desc(NB - 1, 1).wait()

            plsc.subcore_barrier()

            for r in range(ROUNDS):
                cid = t + NSUB * r

                @pl.when(cid < NCHUNK)
                def _():
                    pltpu.sync_copy(agg_sh.at[pl.ds(cid * CH, CH)], dbuf)
                    lax.fori_loop(0, CH, div_row, 0)
                    pltpu.sync_copy(
                        obuf,
                        out_hbm.at[c * npass + ps].at[pl.ds(cid * CH, CH)])

            if ps + 1 < npass:
                plsc.subcore_barrier()

    mesh = plsc.VectorSubcoreMesh(core_axis_name="c", subcore_axis_name="s")
    return pl.kernel(
        body,
        out_type=jax.ShapeDtypeStruct((NCORE * npass, N, DH), jnp.float32),
        mesh=mesh,
        scratch_types=[
            pltpu.VMEM((2 * N,), jnp.float32),
            pltpu.VMEM((NB, BR), jnp.int32),
            pltpu.VMEM((NB, BR), jnp.int32),
            pltpu.VMEM((NB, BR), jnp.float32),
            pltpu.VMEM((BR, DH), jnp.float32),
            pltpu.VMEM((BR, DH), jnp.float32),
            pltpu.VMEM((BR, DH), jnp.float32),
            pltpu.VMEM((BR, DH), jnp.float32),
            pltpu.VMEM((BR, ROW), jnp.float32),
            pltpu.VMEM((BR, ROW), jnp.float32),
            pltpu.VMEM((CH, ROW), jnp.float32),
            pltpu.VMEM((CH, DH), jnp.float32),
            pltpu.VMEM_SHARED((N, ROW), jnp.float32),
            pltpu.SemaphoreType.DMA,
            pltpu.SemaphoreType.DMA,
        ],
        compiler_params=pltpu.CompilerParams(
            needs_layout_passes=False, use_tc_tiling_on_sc=False),
        name=f"sc_gat_p{npass}",
    )


_sc_gat_l1 = _sc_gat_layer(2)
_sc_gat_l2 = _sc_gat_layer(1)


_BN = 400
_GRID = N // _BN


def _k1_body(x_ref, w1_ref, rwt_ref, rb_ref, amat_ref,
             xp_ref, id_ref, al_ref):
    xb = x_ref[...]
    xp = jnp.dot(xb, w1_ref[...], preferred_element_type=jnp.float32)
    for b in range(4):
        xp_ref[b] = xp[:, b * DH:(b + 1) * DH]
    id_ref[...] = (
        jnp.dot(xb, rwt_ref[...], preferred_element_type=jnp.float32)
        + rb_ref[...]
    )
    al_ref[...] = jnp.dot(xp, amat_ref[...], preferred_element_type=jnp.float32)


def _k1(x, w1, rwt, rb, amat):
    return pl.pallas_call(
        _k1_body,
        grid=(_GRID,),
        in_specs=[
            pl.BlockSpec((_BN, IN), lambda i: (i, 0)),
            pl.BlockSpec((IN, 2 * OUT), lambda i: (0, 0)),
            pl.BlockSpec((IN, 2 * OUT), lambda i: (0, 0)),
            pl.BlockSpec((1, 2 * OUT), lambda i: (0, 0)),
            pl.BlockSpec((2 * OUT, 8), lambda i: (0, 0)),
        ],
        out_specs=[
            pl.BlockSpec((4, _BN, DH), lambda i: (0, i, 0)),
            pl.BlockSpec((_BN, 2 * OUT), lambda i: (i, 0)),
            pl.BlockSpec((_BN, 8), lambda i: (i, 0)),
        ],
        out_shape=[
            jax.ShapeDtypeStruct((4, N, DH), jnp.float32),
            jax.ShapeDtypeStruct((N, 2 * OUT), jnp.float32),
            jax.ShapeDtypeStruct((N, 8), jnp.float32),
        ],
    )(x, w1, rwt, rb, amat)


def _stats_body(nparts, p_ref, out_ref, acc):
    i = pl.program_id(0)

    @pl.when(i == 0)
    def _():
        acc[...] = jnp.zeros_like(acc)

    pre = jnp.concatenate([p_ref[b] for b in range(nparts)], axis=-1)
    acc[0:1, :] += jnp.sum(pre, axis=0, keepdims=True)
    acc[1:2, :] += jnp.sum(pre * pre, axis=0, keepdims=True)

    @pl.when(i == pl.num_programs(0) - 1)
    def _():
        out_ref[...] = acc[...]


def _stats(parts):
    nparts = parts.shape[0]
    d = nparts * DH
    return pl.pallas_call(
        functools.partial(_stats_body, nparts),
        grid=(_GRID,),
        in_specs=[pl.BlockSpec((nparts, _BN, DH), lambda i: (0, i, 0))],
        out_specs=pl.BlockSpec((2, d), lambda i: (0, 0)),
        out_shape=jax.ShapeDtypeStruct((2, d), jnp.float32),
        scratch_shapes=[pltpu.VMEM((2, d), jnp.float32)],
    )(parts)


def _k2b_body(p_ref, sums_ref, b1_ref, g_ref, bb_ref, id_ref, w2_ref,
              a2_ref, xp2_ref, al2_ref):
    pre = jnp.concatenate([p_ref[b] for b in range(4)], axis=-1) + b1_ref[...]
    s1 = sums_ref[0:1, :]
    s2 = sums_ref[1:2, :]
    mu = s1 / N + b1_ref[...]
    var = s2 / N - (s1 / N) ** 2
    scale = g_ref[...] / jnp.sqrt(var + 1e-5)
    h = jnp.maximum(scale * (pre - mu) + bb_ref[...], 0.0) + id_ref[...]
    xp2 = jnp.dot(h, w2_ref[...], preferred_element_type=jnp.float32)
    xp2_ref[0] = xp2[:, :DH]
    xp2_ref[1] = xp2[:, DH:]
    al2_ref[...] = jnp.dot(xp2, a2_ref[...], preferred_element_type=jnp.float32)


def _k2b(parts, sums, b1, g, bb, idt, w2, a2mat):
    return pl.pallas_call(
        _k2b_body,
        grid=(_GRID,),
        in_specs=[
            pl.BlockSpec((4, _BN, DH), lambda i: (0, i, 0)),
            pl.BlockSpec((2, 2 * OUT), lambda i: (0, 0)),
            pl.BlockSpec((1, 2 * OUT), lambda i: (0, 0)),
            pl.BlockSpec((1, 2 * OUT), lambda i: (0, 0)),
            pl.BlockSpec((1, 2 * OUT), lambda i: (0, 0)),
            pl.BlockSpec((_BN, 2 * OUT), lambda i: (i, 0)),
            pl.BlockSpec((2 * OUT, OUT), lambda i: (0, 0)),
            pl.BlockSpec((OUT, 8), lambda i: (0, 0)),
        ],
        out_specs=[
            pl.BlockSpec((2, _BN, DH), lambda i: (0, i, 0)),
            pl.BlockSpec((_BN, 8), lambda i: (i, 0)),
        ],
        out_shape=[
            jax.ShapeDtypeStruct((2, N, DH), jnp.float32),
            jax.ShapeDtypeStruct((N, 8), jnp.float32),
        ],
    )(parts, sums, b1, g, bb, idt, w2, a2mat)


def _k4b_body(p_ref, sums_ref, b2_ref, g_ref, bb_ref, out_ref):
    pre = jnp.concatenate([p_ref[0], p_ref[1]], axis=-1) + b2_ref[...]
    s1 = sums_ref[0:1, :]
    s2 = sums_ref[1:2, :]
    mu = s1 / N + b2_ref[...]
    var = s2 / N - (s1 / N) ** 2
    scale = g_ref[...] / jnp.sqrt(var + 1e-5)
    out_ref[...] = jnp.maximum(scale * (pre - mu) + bb_ref[...], 0.0)


def _k4b(parts, sums, b2, g, bb):
    return pl.pallas_call(
        _k4b_body,
        grid=(_GRID,),
        in_specs=[
            pl.BlockSpec((2, _BN, DH), lambda i: (0, i, 0)),
            pl.BlockSpec((2, OUT), lambda i: (0, 0)),
            pl.BlockSpec((1, OUT), lambda i: (0, 0)),
            pl.BlockSpec((1, OUT), lambda i: (0, 0)),
            pl.BlockSpec((1, OUT), lambda i: (0, 0)),
        ],
        out_specs=pl.BlockSpec((_BN, OUT), lambda i: (i, 0)),
        out_shape=jax.ShapeDtypeStruct((N, OUT), jnp.float32),
    )(parts, sums, b2, g, bb)


def kernel(x, edge_index, W1, a_src1, a_dst1, b1, bn1_g, bn1_b, res_W, res_b,
           W2, a_src2, a_dst2, b2, bn2_g, bn2_b):
    f32 = jnp.float32

    loop = jnp.arange(N, dtype=edge_index.dtype)
    src_f = jnp.concatenate([edge_index[0], loop])
    dst_f = jnp.concatenate([edge_index[1], loop])
    pad = jnp.zeros((EPAD - EF,), dtype=edge_index.dtype)
    src_r = jnp.concatenate([src_f, pad]).astype(jnp.int32).reshape(NSUB, NB, BR)
    dst_r = jnp.concatenate([dst_f, pad]).astype(jnp.int32).reshape(NSUB, NB, BR)

    amat1 = jnp.zeros((2 * OUT, 8), f32)
    amat1 = amat1.at[:OUT, 0].set(a_src1[0])
    amat1 = amat1.at[OUT:, 1].set(a_src1[1])
    amat1 = amat1.at[:OUT, 2].set(a_dst1[0])
    amat1 = amat1.at[OUT:, 3].set(a_dst1[1])
    amat2 = jnp.zeros((OUT, 8), f32)
    amat2 = amat2.at[:, 0].set(a_src2[0])
    amat2 = amat2.at[:, 1].set(a_dst2[0])

    xp_parts, identity, al1 = _k1(
        x, W1, res_W.T, res_b.reshape(1, -1), amat1)

    tab1 = jnp.stack([
        jnp.stack([al1[:, 0], al1[:, 2]], axis=1).reshape(-1),
        jnp.stack([al1[:, 1], al1[:, 3]], axis=1).reshape(-1),
    ])

    agg1 = _sc_gat_l1(xp_parts.reshape(4 * N, DH), tab1, src_r, dst_r)

    sums1 = _stats(agg1)
    xp2_parts, al2 = _k2b(
        agg1, sums1, b1.reshape(1, -1), bn1_g.reshape(1, -1),
        bn1_b.reshape(1, -1), identity, W2, amat2)

    tab2_row = jnp.stack([al2[:, 0], al2[:, 1]], axis=1).reshape(-1)
    tab2 = jnp.stack([tab2_row, tab2_row])

    agg2 = _sc_gat_l2(xp2_parts.reshape(2 * N, DH), tab2, src_r, dst_r)

    sums2 = _stats(agg2)
    return _k4b(agg2, sums2, b2.reshape(1, -1), bn2_g.reshape(1, -1),
                bn2_b.reshape(1, -1))
